# baseline (speedup 1.0000x reference)
import dataclasses
import functools

import jax
import jax.numpy as jnp
from jax import lax
from jax.experimental import pallas as pl
from jax.experimental.pallas import tpu as pltpu
from jax.experimental.pallas import tpu_sc as plsc

_NC, _NS = 2, 16
_NW = _NC * _NS
_W = 128


def kernel(token_ids, embeddings):
    batch, seq = token_ids.shape
    halves = []
    hb = batch // 2
    for h in range(2):
        halves.append(_slab(token_ids[h * hb:(h + 1) * hb], embeddings))
    return jnp.concatenate(halves, axis=0)


def _slab(token_ids, embeddings):
    batch, seq = token_ids.shape
    vocab, d = embeddings.shape
    n = batch * seq
    b_per_w = n // _NW
    chunks = b_per_w // _W
    d2 = 2 * d
    table2 = embeddings.reshape(vocab // 2, d2)
    idx = (token_ids >> 1).reshape(_NW, chunks, _W)
    par = (token_ids & 1).reshape(_NW, chunks, _W)

    mesh = plsc.VectorSubcoreMesh(core_axis_name="c", subcore_axis_name="s")
    cp = pltpu.CompilerParams()
    if "needs_layout_passes" in pltpu.CompilerParams.__dataclass_fields__:
        cp = dataclasses.replace(cp, needs_layout_passes=False)

    @functools.partial(
        pl.kernel, mesh=mesh,
        compiler_params=cp,
        out_type=jax.ShapeDtypeStruct((batch, seq, d), jnp.float32),
        scratch_types=[
            pltpu.VMEM((chunks, _W), jnp.int32),
            pltpu.VMEM((chunks, _W), jnp.int32),
            pltpu.VMEM((_W, d2), jnp.float32),
            pltpu.VMEM((_W, d2), jnp.float32),
            pltpu.VMEM((_W, d), jnp.float32),
            pltpu.VMEM((_W, d), jnp.float32),
            pltpu.SemaphoreType.DMA,
            pltpu.SemaphoreType.DMA,
            pltpu.SemaphoreType.DMA,
            pltpu.SemaphoreType.DMA,
        ],
    )
    def _gather(table_hbm, idx_hbm, par_hbm, out_hbm,
                idx_v, par_v, rows0, rows1, sel0, sel1,
                sem0, sem1, wsem0, wsem1):
        out2 = out_hbm.reshape(n, d)
        wid = lax.axis_index("s") * _NC + lax.axis_index("c")
        base = wid * b_per_w
        pltpu.sync_copy(idx_hbm.at[wid], idx_v)
        pltpu.sync_copy(par_hbm.at[wid], par_v)
        bufs = (rows0, rows1)
        sems = (sem0, sem1)
        sels = (sel0, sel1)
        wsems = (wsem0, wsem1)
        pltpu.async_copy(table_hbm.at[idx_v.at[0]], rows0, sem0)

        @pl.loop(0, chunks, step=2)
        def _(g):
            for b in range(2):
                k = g + b
                pltpu.make_async_copy(table_hbm.at[idx_v.at[k]],
                                      bufs[b], sems[b]).wait()

                @pl.when(k + 1 < chunks)
                def _():
                    pltpu.async_copy(table_hbm.at[idx_v.at[k + 1]],
                                     bufs[1 - b], sems[1 - b])

                @pl.when(k >= 2)
                def _():
                    pltpu.make_async_copy(
                        sels[b],
                        out2.at[pl.ds(base + (k - 2) * _W, _W)],
                        wsems[b]).wait()

                buf = bufs[b]
                sel_v = sels[b]
                kvec = jnp.full((16,), k, jnp.int32)
                for r in range(_W):
                    rvec = jnp.full((16,), r, jnp.int32)
                    m16 = plsc.load_gather(par_v, [kvec, rvec]) != 0
                    for j in range(0, d, 16):
                        lo = buf[r, pl.ds(j, 16)]
                        hi = buf[r, pl.ds(d + j, 16)]
                        sel_v[r, pl.ds(j, 16)] = jnp.where(m16, hi, lo)

                pltpu.async_copy(sel_v, out2.at[pl.ds(base + k * _W, _W)],
                                 wsems[b])

        for b in range(2):
            k = chunks - 2 + b
            pltpu.make_async_copy(sels[b],
                                  out2.at[pl.ds(base + k * _W, _W)],
                                  wsems[b]).wait()

    return _gather(table2, idx, par)


# speedup vs baseline: 1.1237x; 1.1237x over previous
import dataclasses
import functools

import jax
import jax.numpy as jnp
from jax import lax
from jax.experimental import pallas as pl
from jax.experimental.pallas import tpu as pltpu
from jax.experimental.pallas import tpu_sc as plsc

_NC, _NS = 2, 16
_NW = _NC * _NS
_W = 128


def kernel(token_ids, embeddings):
    batch, seq = token_ids.shape
    vocab, d = embeddings.shape
    n = batch * seq
    b_per_w = n // _NW
    chunks = b_per_w // _W
    d2 = 2 * d
    table2 = embeddings.reshape(vocab // 2, d2)
    idx = (token_ids >> 1).reshape(_NW, chunks, _W)
    par = (token_ids & 1).reshape(_NW, chunks, _W)

    mesh = plsc.VectorSubcoreMesh(core_axis_name="c", subcore_axis_name="s")
    cp = pltpu.CompilerParams()
    if "needs_layout_passes" in pltpu.CompilerParams.__dataclass_fields__:
        cp = dataclasses.replace(cp, needs_layout_passes=False)

    @functools.partial(
        pl.kernel, mesh=mesh,
        compiler_params=cp,
        out_type=jax.ShapeDtypeStruct((batch, seq, d), jnp.float32),
        scratch_types=[
            pltpu.VMEM((chunks, _W), jnp.int32),
            pltpu.VMEM((chunks, _W), jnp.int32),
            pltpu.VMEM((_W, d2), jnp.float32),
            pltpu.VMEM((_W, d2), jnp.float32),
            pltpu.VMEM((_W, d), jnp.float32),
            pltpu.VMEM((_W, d), jnp.float32),
            pltpu.SemaphoreType.DMA,
            pltpu.SemaphoreType.DMA,
            pltpu.SemaphoreType.DMA,
            pltpu.SemaphoreType.DMA,
        ],
    )
    def _gather(table_hbm, idx_hbm, par_hbm, out_hbm,
                idx_v, par_v, rows0, rows1, sel0, sel1,
                sem0, sem1, wsem0, wsem1):
        out2 = out_hbm.reshape(n, d)
        wid = lax.axis_index("s") * _NC + lax.axis_index("c")
        base = wid * b_per_w
        pltpu.sync_copy(idx_hbm.at[wid], idx_v)
        pltpu.sync_copy(par_hbm.at[wid], par_v)
        bufs = (rows0, rows1)
        sems = (sem0, sem1)
        sels = (sel0, sel1)
        wsems = (wsem0, wsem1)
        pltpu.async_copy(table_hbm.at[idx_v.at[0]], rows0, sem0)

        @pl.loop(0, chunks, step=2)
        def _(g):
            for b in range(2):
                k = g + b
                pltpu.make_async_copy(table_hbm.at[idx_v.at[k]],
                                      bufs[b], sems[b]).wait()

                @pl.when(k + 1 < chunks)
                def _():
                    pltpu.async_copy(table_hbm.at[idx_v.at[k + 1]],
                                     bufs[1 - b], sems[1 - b])

                @pl.when(k >= 2)
                def _():
                    pltpu.make_async_copy(
                        sels[b],
                        out2.at[pl.ds(base + (k - 2) * _W, _W)],
                        wsems[b]).wait()

                buf = bufs[b]
                sel_v = sels[b]
                kvec = jnp.full((16,), k, jnp.int32)
                for r in range(_W):
                    rvec = jnp.full((16,), r, jnp.int32)
                    m16 = plsc.load_gather(par_v, [kvec, rvec]) != 0
                    for j in range(0, d, 16):
                        lo = buf[r, pl.ds(j, 16)]
                        hi = buf[r, pl.ds(d + j, 16)]
                        sel_v[r, pl.ds(j, 16)] = jnp.where(m16, hi, lo)

                pltpu.async_copy(sel_v, out2.at[pl.ds(base + k * _W, _W)],
                                 wsems[b])

        for b in range(2):
            k = chunks - 2 + b
            pltpu.make_async_copy(sels[b],
                                  out2.at[pl.ds(base + k * _W, _W)],
                                  wsems[b]).wait()

    return _gather(table2, idx, par)
